# baseline (speedup 1.0000x reference)
import functools

import jax
import jax.numpy as jnp
from jax import lax
from jax.experimental import pallas as pl
from jax.experimental.pallas import tpu as pltpu
from jax.experimental.pallas import tpu_sc as plsc

_C_PAD = 128


def _tc_body(x_ref, w_ref, cc_ref, sims_ref, bmu_ref, flat_ref, pmi_ref,
             wn_ref, *, uy):
    i = pl.program_id(0)

    @pl.when(i == 0)
    def _prep():
        wv = w_ref[...]
        wn = jnp.sqrt(jnp.sum(wv * wv, axis=1, keepdims=True))
        wn_ref[...] = wv / (wn + 1e-6)

        cc = cc_ref[...]
        denom = jnp.sum(cc, axis=1, keepdims=True)
        cond = cc / (denom + 1e-6)
        prior = jnp.sum(cc, axis=0, keepdims=True)
        prior = prior / (jnp.sum(cc) + 1e-6)
        pmi_ref[...] = jnp.log(cond / (prior + 1e-6) + 1e-6)

    xv = x_ref[...]
    xn = jnp.sqrt(jnp.sum(xv * xv, axis=1, keepdims=True))
    xv = xv / (xn + 1e-6)

    sims = lax.dot_general(
        xv, wn_ref[...],
        dimension_numbers=(((1,), (1,)), ((), ())),
        preferred_element_type=jnp.float32,
    )
    sims_ref[...] = sims

    tq = sims.shape[0]
    flat = jnp.argmax(sims, axis=1).astype(jnp.int32)
    flat2 = flat.reshape(tq, 1)
    flat_ref[...] = flat2
    bmu_ref[...] = jnp.concatenate([flat2 // uy, flat2 % uy], axis=1)


def _sc_gather_make(b, d_pad):
    info = plsc.get_sparse_core_info()
    nw = info.num_cores * info.num_subcores
    b_per_w = b // nw
    mesh = plsc.VectorSubcoreMesh(core_axis_name="c", subcore_axis_name="s")

    @functools.partial(
        pl.kernel, mesh=mesh,
        out_type=jax.ShapeDtypeStruct((b, d_pad), jnp.float32),
        scratch_types=[
            pltpu.VMEM((b_per_w,), jnp.int32),
            pltpu.VMEM((b_per_w, d_pad), jnp.float32),
            pltpu.SemaphoreType.DMA,
        ],
    )
    def gather_k(table_hbm, idx_hbm, out_hbm, idx_v, rows_v, sem):
        wid = lax.axis_index("s") * info.num_cores + lax.axis_index("c")
        base = wid * b_per_w
        pltpu.sync_copy(idx_hbm.at[pl.ds(base, b_per_w)], idx_v)
        pltpu.async_copy(table_hbm.at[idx_v], rows_v, sem).wait()
        pltpu.sync_copy(rows_v, out_hbm.at[pl.ds(base, b_per_w)])

    return gather_k


def kernel(x, som, class_count):
    q, d = x.shape
    ux, uy, _ = som.shape
    k = ux * uy
    c = class_count.shape[-1]
    w = som.reshape(k, d)
    cc = jnp.pad(class_count.reshape(k, c), ((0, 0), (0, _C_PAD - c)))

    tq = 256
    grid = (q // tq,)

    sims, bmu, flat, pmi = pl.pallas_call(
        functools.partial(_tc_body, uy=uy),
        grid=grid,
        in_specs=[
            pl.BlockSpec((tq, d), lambda i: (i, 0)),
            pl.BlockSpec((k, d), lambda i: (0, 0)),
            pl.BlockSpec((k, _C_PAD), lambda i: (0, 0)),
        ],
        out_specs=[
            pl.BlockSpec((tq, k), lambda i: (i, 0)),
            pl.BlockSpec((tq, 2), lambda i: (i, 0)),
            pl.BlockSpec((tq, 1), lambda i: (i, 0)),
            pl.BlockSpec((k, _C_PAD), lambda i: (0, 0)),
        ],
        out_shape=[
            jax.ShapeDtypeStruct((q, k), jnp.float32),
            jax.ShapeDtypeStruct((q, 2), jnp.int32),
            jax.ShapeDtypeStruct((q, 1), jnp.int32),
            jax.ShapeDtypeStruct((k, _C_PAD), jnp.float32),
        ],
        scratch_shapes=[
            pltpu.VMEM((k, d), jnp.float32),
        ],
    )(x, w, cc)

    bmu_pmi = _sc_gather_make(q, _C_PAD)(pmi, flat.reshape(q))[:, :c]
    return sims, bmu, bmu_pmi


# speedup vs baseline: 1.0250x; 1.0250x over previous
import functools

import jax
import jax.numpy as jnp
from jax import lax
from jax.experimental import pallas as pl
from jax.experimental.pallas import tpu as pltpu
from jax.experimental.pallas import tpu_sc as plsc

_C_PAD = 128


def _tc_body(x_ref, w_ref, cc_ref, sims_ref, bmu_ref, flat_ref, pmi_ref,
             wn_ref, *, uy):
    i = pl.program_id(0)

    @pl.when(i == 0)
    def _prep():
        wv = w_ref[...]
        wn = jnp.sqrt(jnp.sum(wv * wv, axis=1, keepdims=True))
        wn_ref[...] = wv / (wn + 1e-6)

        cc = cc_ref[...]
        denom = jnp.sum(cc, axis=1, keepdims=True)
        cond = cc / (denom + 1e-6)
        prior = jnp.sum(cc, axis=0, keepdims=True)
        prior = prior / (jnp.sum(cc) + 1e-6)
        pmi_ref[...] = jnp.log(cond / (prior + 1e-6) + 1e-6)

    xv = x_ref[...]
    xn = jnp.sqrt(jnp.sum(xv * xv, axis=1, keepdims=True))
    xv = xv / (xn + 1e-6)

    sims = lax.dot_general(
        xv, wn_ref[...],
        dimension_numbers=(((1,), (1,)), ((), ())),
        preferred_element_type=jnp.float32,
    )
    sims_ref[...] = sims

    tq = sims.shape[0]
    flat = jnp.argmax(sims, axis=1).astype(jnp.int32)
    flat2 = flat.reshape(tq, 1)
    flat_ref[...] = flat2
    bmu_ref[...] = jnp.concatenate([flat2 // uy, flat2 % uy], axis=1)


def _sc_gather_make(b, d_pad):
    info = plsc.get_sparse_core_info()
    nw = info.num_cores * info.num_subcores
    b_per_w = b // nw
    mesh = plsc.VectorSubcoreMesh(core_axis_name="c", subcore_axis_name="s")

    @functools.partial(
        pl.kernel, mesh=mesh,
        out_type=jax.ShapeDtypeStruct((b, d_pad), jnp.float32),
        scratch_types=[
            pltpu.VMEM((b_per_w,), jnp.int32),
            pltpu.VMEM((b_per_w, d_pad), jnp.float32),
            pltpu.SemaphoreType.DMA,
        ],
    )
    def gather_k(table_hbm, idx_hbm, out_hbm, idx_v, rows_v, sem):
        wid = lax.axis_index("s") * info.num_cores + lax.axis_index("c")
        base = wid * b_per_w
        pltpu.sync_copy(idx_hbm.at[pl.ds(base, b_per_w)], idx_v)
        pltpu.async_copy(table_hbm.at[idx_v], rows_v, sem).wait()
        pltpu.sync_copy(rows_v, out_hbm.at[pl.ds(base, b_per_w)])

    return gather_k


def kernel(x, som, class_count):
    q, d = x.shape
    ux, uy, _ = som.shape
    k = ux * uy
    c = class_count.shape[-1]
    w = som.reshape(k, d)
    cc = jnp.pad(class_count.reshape(k, c), ((0, 0), (0, _C_PAD - c)))

    tq = 512
    grid = (q // tq,)

    sims, bmu, flat, pmi = pl.pallas_call(
        functools.partial(_tc_body, uy=uy),
        grid=grid,
        in_specs=[
            pl.BlockSpec((tq, d), lambda i: (i, 0)),
            pl.BlockSpec((k, d), lambda i: (0, 0)),
            pl.BlockSpec((k, _C_PAD), lambda i: (0, 0)),
        ],
        out_specs=[
            pl.BlockSpec((tq, k), lambda i: (i, 0)),
            pl.BlockSpec((tq, 2), lambda i: (i, 0)),
            pl.BlockSpec((tq, 1), lambda i: (i, 0)),
            pl.BlockSpec((k, _C_PAD), lambda i: (0, 0)),
        ],
        out_shape=[
            jax.ShapeDtypeStruct((q, k), jnp.float32),
            jax.ShapeDtypeStruct((q, 2), jnp.int32),
            jax.ShapeDtypeStruct((q, 1), jnp.int32),
            jax.ShapeDtypeStruct((k, _C_PAD), jnp.float32),
        ],
        scratch_shapes=[
            pltpu.VMEM((k, d), jnp.float32),
        ],
    )(x, w, cc)

    bmu_pmi = _sc_gather_make(q, _C_PAD)(pmi, flat.reshape(q))[:, :c]
    return sims, bmu, bmu_pmi
